# baseline (speedup 1.0000x reference)
import jax
import jax.numpy as jnp
from jax.experimental import pallas as pl


_B = 16384
_D = 64
_ROWS = 2048


def _dot_rows_kernel(x_ref, o_ref):
    gu = x_ref[0]
    fi = x_ref[1]
    o_ref[...] = jnp.sum(gu * fi, axis=1, keepdims=True)


def kernel(inputs):
    out = pl.pallas_call(
        _dot_rows_kernel,
        grid=(_B // _ROWS,),
        in_specs=[pl.BlockSpec((2, _ROWS, _D), lambda i: (0, i, 0))],
        out_specs=pl.BlockSpec((_ROWS, 1), lambda i: (i, 0)),
        out_shape=jax.ShapeDtypeStruct((_B, 1), jnp.float32),
    )(inputs)
    return out.reshape(_B)
